# baseline (speedup 1.0000x reference)
import functools

import jax
import jax.numpy as jnp
from jax import lax
from jax.experimental import pallas as pl
from jax.experimental.pallas import tpu as pltpu
from jax.experimental.pallas import tpu_sc as plsc

VOCAB = 1_000_000
D = 64
B_TOKENS = 4096
SEQ = 200
B = B_TOKENS * SEQ

NC = 2
NS = 16
NW = NC * NS
B_PER_W = B // NW
BLK = 128
NBLK = B_PER_W // BLK

VT = VOCAB // 128
VT_MAIN = VT // NW
VT_EXTRA = VT - NW * VT_MAIN
PAIRS = VOCAB // 2


def _lanes():
    iota = lax.iota(jnp.int32, 16)
    return iota, iota >> 1, (iota & 1) * 64


def _repack_body(wt_hbm, tail_hbm, scr_hbm, ibuf, obuf, *sems):
    wid = lax.axis_index("s") * NC + lax.axis_index("c")
    isems = sems[:2]
    osems = sems[2:]
    _, half, odd64 = _lanes()

    nvb = VT_MAIN + jnp.where(wid < VT_EXTRA, 1, 0)

    def vb_of(i):
        return wid + i * NW

    def load(i, b):
        return pltpu.make_async_copy(
            wt_hbm.at[:, pl.ds(vb_of(i) * 128, 128)], ibuf.at[b], isems[b])

    def store(i, b):
        return pltpu.make_async_copy(
            obuf.at[b, :, pl.ds(0, 128)],
            scr_hbm.at[pl.ds(vb_of(i) * 64, 64)], osems[b])

    rows_k = [8 * k + half for k in range(8)]

    def transpose(b, kmax):
        src = ibuf.at[b]
        dsts = [obuf.at[b, pl.ds(8 * k, 8)] for k in range(kmax)]

        for d in range(64):
            cols = odd64 + d
            for k in range(kmax):
                vals = src[d, pl.ds(16 * k, 16)]
                plsc.store_scatter(dsts[k], [half, cols], vals)

    load(0, 0).start()

    @pl.loop(0, (VT_MAIN + 2) // 2)
    def _(s):
        for b in range(2):
            i = 2 * s + b

            @pl.when(i < nvb)
            def _():
                load(i, b).wait()

                @pl.when(i + 1 < nvb)
                def _():
                    load(i + 1, 1 - b).start()

                @pl.when(i >= 2)
                def _():
                    store(i - 2, b).wait()

                transpose(b, 8)
                store(i, b).start()

    for b in range(2):
        last = nvb - 1 - ((nvb - 1 - b) % 2)

        @pl.when(last >= 0)
        def _():
            store(last, b).wait()

    @pl.when(wid == NW - 1)
    def _():
        pltpu.sync_copy(tail_hbm, ibuf.at[0, pl.ds(0, 32)])
        pltpu.sync_copy(ibuf.at[0, pl.ds(0, 32)],
                        scr_hbm.at[pl.ds(VT * 64, 32)])


def _compact(src, dst, pv_k):
    for k in range(8):
        for i in range(16):
            t = 16 * k + i
            p = pv_k[k][i]
            for m in range(4):
                dst[t, pl.ds(16 * m, 16)] = src[t, pl.ds(p + 16 * m, 16)]


def _gather_body(scr_hbm, idx_hbm, par_hbm, out_hbm,
                 idx_v, par_v, gbuf, obuf, *sems):
    wid = lax.axis_index("s") * NC + lax.axis_index("c")
    base = wid * B_PER_W
    gsems = sems[:2]
    osems = sems[2:]
    iota, _, _ = _lanes()

    pltpu.sync_copy(idx_hbm.at[wid], idx_v)
    pltpu.sync_copy(par_hbm.at[wid], par_v)

    def gather(j, b):
        return pltpu.make_async_copy(
            scr_hbm.at[idx_v.at[j]], gbuf.at[b], gsems[b])

    def writeback(j, b):
        return pltpu.make_async_copy(
            obuf.at[b], out_hbm.at[pl.ds(base + j * BLK, BLK)], osems[b])

    def compact(j, b):
        pv_k = [par_v[j, pl.ds(16 * k, 16)] for k in range(8)]
        _compact(gbuf.at[b], obuf.at[b], pv_k)

    gather(0, 0).start()

    @pl.loop(0, NBLK // 2)
    def _(s):
        for b in range(2):
            j = 2 * s + b
            gather(j, b).wait()

            @pl.when(j + 1 < NBLK)
            def _():
                gather(j + 1, 1 - b).start()

            @pl.when(j >= 2)
            def _():
                writeback(j - 2, b).wait()

            compact(j, b)
            writeback(j, b).start()

    writeback(NBLK - 2, 0).wait()
    writeback(NBLK - 1, 1).wait()


@functools.cache
def _build():
    mesh = plsc.VectorSubcoreMesh(
        core_axis_name="c", subcore_axis_name="s",
        num_cores=NC, num_subcores=NS)
    params = pltpu.CompilerParams(use_tc_tiling_on_sc=True,
                                  needs_layout_passes=False)
    repack = pl.kernel(
        _repack_body,
        out_type=jax.ShapeDtypeStruct((PAIRS, 128), jnp.float32),
        mesh=mesh,
        compiler_params=params,
        scratch_types=[
            pltpu.VMEM((2, 64, 128), jnp.float32),
            pltpu.VMEM((2, 64, 130), jnp.float32),
        ] + [pltpu.SemaphoreType.DMA] * 4,
    )
    gather = pl.kernel(
        _gather_body,
        out_type=jax.ShapeDtypeStruct((B, D), jnp.float32),
        mesh=mesh,
        compiler_params=params,
        scratch_types=[
            pltpu.VMEM((NBLK, BLK), jnp.int32),
            pltpu.VMEM((NBLK, BLK), jnp.int32),
            pltpu.VMEM((2, BLK, 128), jnp.float32),
            pltpu.VMEM((2, BLK, D), jnp.float32),
        ] + [pltpu.SemaphoreType.DMA] * 4,
    )
    return repack, gather


def kernel(token_ids, weight):
    repack, gather = _build()
    ids = token_ids.reshape(-1).astype(jnp.int32)
    idx2 = (ids >> 1).reshape(NW, NBLK, BLK)
    par64 = ((ids & 1) * 64).reshape(NW, NBLK, BLK)
    tail = weight[VT * 128:].reshape(32, 128)
    scratch = repack(weight.T, tail)
    out = gather(scratch, idx2, par64)
    return out.reshape(B_TOKENS, SEQ, D)


# speedup vs baseline: 1.5226x; 1.5226x over previous
import functools

import jax
import jax.numpy as jnp
from jax import lax
from jax.experimental import pallas as pl
from jax.experimental.pallas import tpu as pltpu
from jax.experimental.pallas import tpu_sc as plsc

VOCAB = 1_000_000
D = 64
B_TOKENS = 4096
SEQ = 200
B = B_TOKENS * SEQ

NC = 2
NS = 16
NW = NC * NS
B_PER_W = B // NW
BLK = 128
NBLK = B_PER_W // BLK

NBUF = 8
AHEAD = 4


def _body(idx_hbm, table_hbm, out_hbm, idx_v, rows_v, *sems):
    wid = lax.axis_index("s") * NC + lax.axis_index("c")
    base = wid * B_PER_W

    pltpu.sync_copy(idx_hbm.at[wid], idx_v)

    gsems = sems[:NBUF]
    osems = sems[NBUF:]

    def gather(j, b):
        return pltpu.make_async_copy(
            table_hbm.at[idx_v.at[j]], rows_v.at[b], gsems[b])

    def writeback(j, b):
        return pltpu.make_async_copy(
            rows_v.at[b], out_hbm.at[pl.ds(base + j * BLK, BLK)], osems[b])

    for g in range(AHEAD):
        gather(g, g).start()

    @pl.loop(0, NBLK // NBUF)
    def _(s):
        for b in range(NBUF):
            j = s * NBUF + b
            gather(j, b).wait()
            writeback(j, b).start()

            @pl.when(j >= AHEAD)
            def _():
                writeback(j - AHEAD, (b - AHEAD) % NBUF).wait()

            @pl.when(j + AHEAD < NBLK)
            def _():
                gather(j + AHEAD, (b + AHEAD) % NBUF).start()

    for j in range(NBLK - AHEAD, NBLK):
        writeback(j, j % NBUF).wait()


@functools.cache
def _build():
    return pl.kernel(
        _body,
        out_type=jax.ShapeDtypeStruct((B, D), jnp.float32),
        mesh=plsc.VectorSubcoreMesh(
            core_axis_name="c", subcore_axis_name="s",
            num_cores=NC, num_subcores=NS),
        compiler_params=pltpu.CompilerParams(use_tc_tiling_on_sc=False),
        scratch_types=[
            pltpu.VMEM((NBLK, BLK), jnp.int32),
            pltpu.VMEM((NBUF, BLK, D), jnp.float32),
        ] + [pltpu.SemaphoreType.DMA] * (2 * NBUF),
    )


def kernel(token_ids, weight):
    idx = token_ids.reshape(NW, NBLK, BLK).astype(jnp.int32)
    out = _build()(idx, weight)
    return out.reshape(B_TOKENS, SEQ, D)
